# Initial kernel scaffold; baseline (speedup 1.0000x reference)
import jax
import jax.numpy as jnp
from jax.experimental import pallas as pl


def kernel(x, h, edge_index, batch, Wemb, bemb, Wres0, Wmsg0, bmsg0, Wupd0, bupd0, Wres1, Wmsg1, bmsg1, Wupd1, bupd1):
    raise NotImplementedError("write your pallas kernel here")


# speedup vs baseline: 3.5047x; 3.5047x over previous
import functools

import jax
import jax.numpy as jnp
from jax import lax
from jax.experimental import pallas as pl
from jax.experimental.pallas import tpu as pltpu
from jax.experimental.pallas import tpu_sc as plsc

N = 10000
E = 320000
H = 128
NC = 2
NS = 16
NW = NC * NS
EPW = E // NW
C = 80
NCHUNK = EPW // C
Z = 624

f32 = jnp.float32

BLK = 1000
GRID = N // BLK


def _node_pre_body(h_ref, Wemb_ref, bemb_ref, Wa_ref, Wb_ref, bmsg_ref,
                   ne_ref, A_ref, B_ref):
    ne = jnp.dot(h_ref[...], Wemb_ref[...], preferred_element_type=f32)
    ne = ne + bemb_ref[...]
    ne_ref[...] = ne
    A_ref[...] = jnp.dot(ne, Wa_ref[...], preferred_element_type=f32) + bmsg_ref[...]
    B_ref[...] = jnp.dot(ne, Wb_ref[...], preferred_element_type=f32)


def _update_pre_body(ne_ref, p_ref, Wua_ref, Wub_ref, bupd_ref, Wres_ref,
                     Wa_ref, Wb_ref, bmsg_ref, ne1_ref, A_ref, B_ref):
    aggr = p_ref[0] + p_ref[1]
    ne = ne_ref[...]
    upd = jnp.dot(ne, Wua_ref[...], preferred_element_type=f32)
    upd = upd + jnp.dot(aggr, Wub_ref[...], preferred_element_type=f32)
    upd = jnp.maximum(upd + bupd_ref[...], 0.0)
    ne1 = jnp.dot(ne, Wres_ref[...], preferred_element_type=f32) + upd
    ne1_ref[...] = ne1
    A_ref[...] = jnp.dot(ne1, Wa_ref[...], preferred_element_type=f32) + bmsg_ref[...]
    B_ref[...] = jnp.dot(ne1, Wb_ref[...], preferred_element_type=f32)


def _final_body(ne_ref, p_ref, Wua_ref, Wub_ref, bupd_ref, Wres_ref, out_ref):
    aggr = p_ref[0] + p_ref[1]
    ne = ne_ref[...]
    upd = jnp.dot(ne, Wua_ref[...], preferred_element_type=f32)
    upd = upd + jnp.dot(aggr, Wub_ref[...], preferred_element_type=f32)
    upd = jnp.maximum(upd + bupd_ref[...], 0.0)
    out_ref[...] = jnp.dot(ne, Wres_ref[...], preferred_element_type=f32) + upd


_row_spec = pl.BlockSpec((BLK, H), lambda i: (i, 0))
_w_spec = pl.BlockSpec((H, H), lambda i: (0, 0))
_b_spec = pl.BlockSpec((1, H), lambda i: (0, 0))
_p_spec = pl.BlockSpec((NC, BLK, H), lambda i: (0, i, 0))
_nh = jax.ShapeDtypeStruct((N, H), f32)

_node_pre = pl.pallas_call(
    _node_pre_body,
    grid=(GRID,),
    in_specs=[_row_spec, _w_spec, _b_spec, _w_spec, _w_spec, _b_spec],
    out_specs=[_row_spec, _row_spec, _row_spec],
    out_shape=[_nh, _nh, _nh],
)

_update_pre = pl.pallas_call(
    _update_pre_body,
    grid=(GRID,),
    in_specs=[_row_spec, _p_spec, _w_spec, _w_spec, _b_spec, _w_spec,
              _w_spec, _w_spec, _b_spec],
    out_specs=[_row_spec, _row_spec, _row_spec],
    out_shape=[_nh, _nh, _nh],
)

_final = pl.pallas_call(
    _final_body,
    grid=(GRID,),
    in_specs=[_row_spec, _p_spec, _w_spec, _w_spec, _b_spec, _w_spec],
    out_specs=_row_spec,
    out_shape=_nh,
)


def _edge_body(xT_hbm, row_hbm, col_hbm, A_hbm, B_hbm, wd_hbm, zeros_hbm,
               out_hbm, xT_v, wd_v, ri_v, ci_v, dist_v, a_v, b_v, aggr_sh,
               sem_a, sem_b):
    c_id = lax.axis_index("c")
    s_id = lax.axis_index("s")
    wid = c_id * NS + s_id
    base0 = pl.multiple_of(wid * EPW, 16)

    pltpu.sync_copy(xT_hbm, xT_v)
    pltpu.sync_copy(wd_hbm, wd_v)
    zbase = pl.multiple_of(s_id * Z, 8)
    pltpu.sync_copy(zeros_hbm.at[pl.ds(zbase, Z)], aggr_sh.at[pl.ds(zbase, Z)])

    @pl.when(s_id == 0)
    def _zero_tail():
        pltpu.sync_copy(zeros_hbm.at[pl.ds(NS * Z, N - NS * Z)],
                        aggr_sh.at[pl.ds(NS * Z, N - NS * Z)])

    plsc.subcore_barrier()

    wd_regs = [wd_v[pl.ds(s * 16, 16)] for s in range(8)]
    zero16 = jnp.zeros((16,), f32)

    def chunk_body(j, carry):
        base = pl.multiple_of(base0 + j * C, 16)
        pltpu.sync_copy(row_hbm.at[pl.ds(base, C)], ri_v)
        pltpu.sync_copy(col_hbm.at[pl.ds(base, C)], ci_v)
        ga = pltpu.async_copy(A_hbm.at[ri_v], a_v, sem_a)
        gb = pltpu.async_copy(B_hbm.at[ci_v], b_v, sem_b)

        for g in range(C // 16):
            ri = ri_v[pl.ds(g * 16, 16)]
            ci = ci_v[pl.ds(g * 16, 16)]
            d = zero16
            for k in range(3):
                xi = plsc.load_gather(xT_v, [ri + (k * N)])
                xj = plsc.load_gather(xT_v, [ci + (k * N)])
                df = xi - xj
                d = d + df * df
            dist_v[pl.ds(g * 16, 16)] = d
        ga.wait()
        gb.wait()

        def edge_body(e, carry2):
            dv = plsc.load_gather(dist_v, [jnp.zeros((16,), jnp.int32) + e])
            for s in range(8):
                sl = pl.ds(s * 16, 16)
                m = a_v[e, sl] + b_v[e, sl] + dv * wd_regs[s]
                a_v[e, sl] = jnp.maximum(m, 0.0)
            return carry2

        lax.fori_loop(0, C, edge_body, 0)
        pltpu.sync_copy(a_v, aggr_sh.at[ci_v], add=True)
        return carry

    lax.fori_loop(0, NCHUNK, chunk_body, 0)

    plsc.subcore_barrier()
    obase = pl.multiple_of(s_id * Z, 8)
    pltpu.sync_copy(aggr_sh.at[pl.ds(obase, Z)],
                    out_hbm.at[c_id, pl.ds(obase, Z)])

    @pl.when(s_id == 0)
    def _out_tail():
        pltpu.sync_copy(aggr_sh.at[pl.ds(NS * Z, N - NS * Z)],
                        out_hbm.at[c_id, pl.ds(NS * Z, N - NS * Z)])


@functools.cache
def _get_edge_pass():
    return pl.kernel(
        _edge_body,
        out_type=jax.ShapeDtypeStruct((NC, N, H), f32),
        mesh=plsc.VectorSubcoreMesh(core_axis_name="c", subcore_axis_name="s",
                                    num_cores=NC, num_subcores=NS),
        compiler_params=pltpu.CompilerParams(needs_layout_passes=False),
        scratch_types=[
        pltpu.VMEM((3 * N,), f32),
        pltpu.VMEM((H,), f32),
        pltpu.VMEM((C,), jnp.int32),
        pltpu.VMEM((C,), jnp.int32),
        pltpu.VMEM((C,), f32),
        pltpu.VMEM((C, H), f32),
        pltpu.VMEM((C, H), f32),
        pltpu.VMEM_SHARED((N, H), f32),
        pltpu.SemaphoreType.DMA,
        pltpu.SemaphoreType.DMA,
        ],
    )


def kernel(x, h, edge_index, batch, Wemb, bemb, Wres0, Wmsg0, bmsg0, Wupd0,
           bupd0, Wres1, Wmsg1, bmsg1, Wupd1, bupd1):
    del batch
    row = edge_index[0]
    col = edge_index[1]
    xT = x.T.reshape(3 * N).astype(f32)
    zeros = jnp.zeros((N, H), f32)

    edge_pass = _get_edge_pass()
    ne, A0, B0 = _node_pre(h, Wemb, bemb[None, :], Wmsg0[:H], Wmsg0[H:2 * H],
                           bmsg0[None, :])
    p0 = edge_pass(xT, row, col, A0, B0, Wmsg0[2 * H], zeros)
    ne1, A1, B1 = _update_pre(ne, p0, Wupd0[:H], Wupd0[H:], bupd0[None, :],
                              Wres0, Wmsg1[:H], Wmsg1[H:2 * H], bmsg1[None, :])
    p1 = edge_pass(xT, row, col, A1, B1, Wmsg1[2 * H], zeros)
    out = _final(ne1, p1, Wupd1[:H], Wupd1[H:], bupd1[None, :], Wres1)
    return out
